# baseline (speedup 1.0000x reference)
import jax
import jax.numpy as jnp
from jax import lax
from jax.experimental import pallas as pl
from jax.experimental.pallas import tpu as pltpu
from jax.experimental.pallas import tpu_sc as plsc

_N = 8192
_D = 256
_SPLIT = 5632
_NC = 2
_NS = 16
_NW = _NC * _NS
_RW = (_N - _SPLIT) // _NW
_RW2 = _N // _NW
_B = 16
_RING = 6
_BT = 704
_GT = _SPLIT // _BT


def _sc_body(emb_hbm, s1_hbm, s2_hbm, w_hbm, shift_hbm, out_hbm,
             buf_v, idxe_v, w_vt, sh_vt, isems, osems):
    wid = lax.axis_index("s") * _NC + lax.axis_index("c")
    base0 = _SPLIT + wid * _RW
    nb = _RW // _B
    assert nb <= _RING

    for b in range(nb):
        r0 = base0 + b * _B
        slot = b % _RING
        pltpu.async_copy(emb_hbm.at[pl.ds(r0, _B)],
                         buf_v.at[slot * 4 + 0], isems.at[slot])
        pltpu.async_copy(s1_hbm.at[pl.ds(r0, _B)],
                         buf_v.at[slot * 4 + 1], isems.at[slot])
        pltpu.async_copy(s2_hbm.at[pl.ds(r0, _B)],
                         buf_v.at[slot * 4 + 2], isems.at[slot])

    st1 = pltpu.async_copy(w_hbm, w_vt.at[pl.ds(0, 3)], osems.at[0])
    st2 = pltpu.async_copy(shift_hbm, sh_vt.at[pl.ds(0, 1)], osems.at[0])
    st1.wait()
    st2.wait()
    wload = w_vt[...]
    w0r = wload[0]
    w1r = wload[1]
    w2r = wload[2]
    shift = sh_vt[...][0]

    iot = lax.iota(jnp.int32, 16)
    wv = jnp.where(iot == 0, w0r,
                   jnp.where(iot == 1, w1r,
                             jnp.where(iot == 2, w2r, -1e30)))
    mx = jnp.maximum(jnp.maximum(w0r, w1r), w2r)
    ev = jnp.exp(wv - mx)
    ssum = ev[0] + ev[1] + ev[2]
    wn = ev / ssum
    w0 = wn[0]
    w1 = wn[1]
    w2 = wn[2]

    def idx_gather(r0, slot):
        for j in range(_B // 16):
            idxe_v[slot, pl.ds(j * 16, 16)] = jnp.clip(
                iot + (r0 + j * 16) + shift, 0, _N - 1)
        return pltpu.async_copy(emb_hbm.at[idxe_v.at[slot]],
                                buf_v.at[slot * 4 + 0], isems.at[slot])

    def wait_inputs(r0, slot):
        pltpu.make_async_copy(s1_hbm.at[pl.ds(r0, _B)],
                              buf_v.at[slot * 4 + 1], isems.at[slot]).wait()
        pltpu.make_async_copy(s1_hbm.at[pl.ds(r0, _B)],
                              buf_v.at[slot * 4 + 2], isems.at[slot]).wait()
        pltpu.make_async_copy(s1_hbm.at[pl.ds(r0, _B)],
                              buf_v.at[slot * 4 + 0], isems.at[slot]).wait()

    def compute(r0, slot):
        def comp(i, carry):
            for j in range(_D // 16):
                sl = pl.ds(j * 16, 16)
                buf_v[slot * 4 + 3, i, sl] = (
                    w0 * buf_v[slot * 4 + 0, i, sl]
                    + w1 * buf_v[slot * 4 + 1, i, sl]
                    + w2 * buf_v[slot * 4 + 2, i, sl])
            return carry

        lax.fori_loop(0, _B, comp, 0)
        return pltpu.async_copy(
            buf_v.at[slot * 4 + 3], out_hbm.at[pl.ds(r0, _B)],
            osems.at[slot])

    def main_blk(b, carry):
        slot = lax.rem(b, _RING)
        r0 = base0 + b * _B
        wait_inputs(r0, slot)
        compute(r0, slot)
        return carry

    lax.fori_loop(0, nb, main_blk, 0)

    def drain_wb(b, carry):
        slot = lax.rem(b, _RING)
        r0 = base0 + b * _B
        pltpu.make_async_copy(buf_v.at[slot * 4 + 3],
                              out_hbm.at[pl.ds(r0, _B)],
                              osems.at[slot]).wait()
        return carry

    lax.fori_loop(0, nb, drain_wb, 0)

    @pl.when(shift != 0)
    def _slow_full_coverage():
        base2 = wid * _RW2

        def blk(b, carry):
            r0 = base2 + b * _B
            pltpu.async_copy(s1_hbm.at[pl.ds(r0, _B)],
                             buf_v.at[1], isems.at[0])
            pltpu.async_copy(s2_hbm.at[pl.ds(r0, _B)],
                             buf_v.at[2], isems.at[0])
            idx_gather(r0, 0)
            wait_inputs(r0, 0)
            compute(r0, 0).wait()
            return carry

        lax.fori_loop(0, _RW2 // _B, blk, 0)


def _run_sc(emb_table, sinusoidal_1, sinusoidal_2, w3, shift1):
    f = pl.kernel(
        _sc_body,
        out_type=jax.ShapeDtypeStruct((_N, _D), jnp.float32),
        mesh=plsc.VectorSubcoreMesh(core_axis_name="c", subcore_axis_name="s"),
        scratch_types=[
            pltpu.VMEM((4 * _RING, _B, _D), jnp.float32),
            pltpu.VMEM((_RING, _B), jnp.int32),
            pltpu.VMEM((16,), jnp.float32),
            pltpu.VMEM((16,), jnp.int32),
            pltpu.SemaphoreType.DMA((_RING,)),
            pltpu.SemaphoreType.DMA((_RING,)),
        ],
    )
    return f(emb_table, sinusoidal_1, sinusoidal_2, w3, shift1)


def _tc_body(w_sm, emb_v, s1_v, s2_v, out_v):
    w0r = w_sm[0]
    w1r = w_sm[1]
    w2r = w_sm[2]
    iot = lax.iota(jnp.int32, 128)
    wv = jnp.where(iot == 0, w0r,
                   jnp.where(iot == 1, w1r,
                             jnp.where(iot == 2, w2r, -1e30)))
    mx = jnp.maximum(jnp.maximum(w0r, w1r), w2r)
    ev = jnp.exp(wv - mx)
    wn = ev / (ev[0] + ev[1] + ev[2])
    out_v[...] = (wn[0] * emb_v[...] + wn[1] * s1_v[...]
                  + wn[2] * s2_v[...])


def _run_tc(emb_table, sinusoidal_1, sinusoidal_2, w3):
    return pl.pallas_call(
        _tc_body,
        grid=(_GT,),
        in_specs=[
            pl.BlockSpec(memory_space=pltpu.SMEM),
            pl.BlockSpec((_BT, _D), lambda i: (i, 0)),
            pl.BlockSpec((_BT, _D), lambda i: (i, 0)),
            pl.BlockSpec((_BT, _D), lambda i: (i, 0)),
        ],
        out_specs=pl.BlockSpec((_BT, _D), lambda i: (i, 0)),
        out_shape=jax.ShapeDtypeStruct((_SPLIT, _D), jnp.float32),
        compiler_params=pltpu.CompilerParams(
            dimension_semantics=("arbitrary",)),
    )(w3, emb_table, sinusoidal_1, sinusoidal_2)


@jax.jit
def _run(emb_table, sinusoidal_1, sinusoidal_2, w3, shift1):
    sc_out = _run_sc(emb_table, sinusoidal_1, sinusoidal_2, w3, shift1)
    tc_out = _run_tc(emb_table, sinusoidal_1, sinusoidal_2, w3)
    return lax.cond(
        shift1[0] == 0,
        lambda: lax.dynamic_update_slice(sc_out, tc_out, (0, 0)),
        lambda: sc_out,
    )


def kernel(emb_table, sinusoidal_1, sinusoidal_2, mixing_weights, seq_len):
    shift1 = jnp.reshape(jnp.asarray(seq_len, jnp.int32) - _N, (1,))
    return _run(emb_table, sinusoidal_1, sinusoidal_2,
                mixing_weights.astype(jnp.float32), shift1)


# speedup vs baseline: 1.0367x; 1.0367x over previous
import jax
import jax.numpy as jnp
from jax import lax
from jax.experimental import pallas as pl
from jax.experimental.pallas import tpu as pltpu
from jax.experimental.pallas import tpu_sc as plsc

_N = 8192
_D = 256
_SPLIT = 5632
_NC = 2
_NS = 16
_NW = _NC * _NS
_RW = (_N - _SPLIT) // _NW
_RW2 = _N // _NW
_B = 16
_RING = 6
_BT = 1408
_GT = _SPLIT // _BT


def _sc_body(emb_hbm, s1_hbm, s2_hbm, w_hbm, shift_hbm, out_hbm,
             buf_v, idxe_v, w_vt, sh_vt, isems, osems):
    wid = lax.axis_index("s") * _NC + lax.axis_index("c")
    base0 = _SPLIT + wid * _RW
    nb = _RW // _B
    assert nb <= _RING

    for b in range(nb):
        r0 = base0 + b * _B
        slot = b % _RING
        pltpu.async_copy(emb_hbm.at[pl.ds(r0, _B)],
                         buf_v.at[slot * 4 + 0], isems.at[slot])
        pltpu.async_copy(s1_hbm.at[pl.ds(r0, _B)],
                         buf_v.at[slot * 4 + 1], isems.at[slot])
        pltpu.async_copy(s2_hbm.at[pl.ds(r0, _B)],
                         buf_v.at[slot * 4 + 2], isems.at[slot])

    st1 = pltpu.async_copy(w_hbm, w_vt.at[pl.ds(0, 3)], osems.at[0])
    st2 = pltpu.async_copy(shift_hbm, sh_vt.at[pl.ds(0, 1)], osems.at[0])
    st1.wait()
    st2.wait()
    wload = w_vt[...]
    w0r = wload[0]
    w1r = wload[1]
    w2r = wload[2]
    shift = sh_vt[...][0]

    iot = lax.iota(jnp.int32, 16)
    wv = jnp.where(iot == 0, w0r,
                   jnp.where(iot == 1, w1r,
                             jnp.where(iot == 2, w2r, -1e30)))
    mx = jnp.maximum(jnp.maximum(w0r, w1r), w2r)
    ev = jnp.exp(wv - mx)
    ssum = ev[0] + ev[1] + ev[2]
    wn = ev / ssum
    w0 = wn[0]
    w1 = wn[1]
    w2 = wn[2]

    def idx_gather(r0, slot):
        for j in range(_B // 16):
            idxe_v[slot, pl.ds(j * 16, 16)] = jnp.clip(
                iot + (r0 + j * 16) + shift, 0, _N - 1)
        return pltpu.async_copy(emb_hbm.at[idxe_v.at[slot]],
                                buf_v.at[slot * 4 + 0], isems.at[slot])

    def wait_inputs(r0, slot):
        pltpu.make_async_copy(s1_hbm.at[pl.ds(r0, _B)],
                              buf_v.at[slot * 4 + 1], isems.at[slot]).wait()
        pltpu.make_async_copy(s1_hbm.at[pl.ds(r0, _B)],
                              buf_v.at[slot * 4 + 2], isems.at[slot]).wait()
        pltpu.make_async_copy(s1_hbm.at[pl.ds(r0, _B)],
                              buf_v.at[slot * 4 + 0], isems.at[slot]).wait()

    def compute(r0, slot):
        def comp(i, carry):
            for j in range(_D // 16):
                sl = pl.ds(j * 16, 16)
                buf_v[slot * 4 + 3, i, sl] = (
                    w0 * buf_v[slot * 4 + 0, i, sl]
                    + w1 * buf_v[slot * 4 + 1, i, sl]
                    + w2 * buf_v[slot * 4 + 2, i, sl])
            return carry

        lax.fori_loop(0, _B, comp, 0)
        return pltpu.async_copy(
            buf_v.at[slot * 4 + 3], out_hbm.at[pl.ds(r0, _B)],
            osems.at[slot])

    def main_blk(b, carry):
        slot = lax.rem(b, _RING)
        r0 = base0 + b * _B
        wait_inputs(r0, slot)
        compute(r0, slot)
        return carry

    lax.fori_loop(0, nb, main_blk, 0)

    def drain_wb(b, carry):
        slot = lax.rem(b, _RING)
        r0 = base0 + b * _B
        pltpu.make_async_copy(buf_v.at[slot * 4 + 3],
                              out_hbm.at[pl.ds(r0, _B)],
                              osems.at[slot]).wait()
        return carry

    lax.fori_loop(0, nb, drain_wb, 0)

    @pl.when(shift != 0)
    def _slow_full_coverage():
        base2 = wid * _RW2

        def blk(b, carry):
            r0 = base2 + b * _B
            pltpu.async_copy(s1_hbm.at[pl.ds(r0, _B)],
                             buf_v.at[1], isems.at[0])
            pltpu.async_copy(s2_hbm.at[pl.ds(r0, _B)],
                             buf_v.at[2], isems.at[0])
            idx_gather(r0, 0)
            wait_inputs(r0, 0)
            compute(r0, 0).wait()
            return carry

        lax.fori_loop(0, _RW2 // _B, blk, 0)


def _run_sc(emb_table, sinusoidal_1, sinusoidal_2, w3, shift1):
    f = pl.kernel(
        _sc_body,
        out_type=jax.ShapeDtypeStruct((_N, _D), jnp.float32),
        mesh=plsc.VectorSubcoreMesh(core_axis_name="c", subcore_axis_name="s"),
        scratch_types=[
            pltpu.VMEM((4 * _RING, _B, _D), jnp.float32),
            pltpu.VMEM((_RING, _B), jnp.int32),
            pltpu.VMEM((16,), jnp.float32),
            pltpu.VMEM((16,), jnp.int32),
            pltpu.SemaphoreType.DMA((_RING,)),
            pltpu.SemaphoreType.DMA((_RING,)),
        ],
    )
    return f(emb_table, sinusoidal_1, sinusoidal_2, w3, shift1)


def _tc_body(w_sm, emb_v, s1_v, s2_v, out_v):
    w0r = w_sm[0]
    w1r = w_sm[1]
    w2r = w_sm[2]
    iot = lax.iota(jnp.int32, 128)
    wv = jnp.where(iot == 0, w0r,
                   jnp.where(iot == 1, w1r,
                             jnp.where(iot == 2, w2r, -1e30)))
    mx = jnp.maximum(jnp.maximum(w0r, w1r), w2r)
    ev = jnp.exp(wv - mx)
    wn = ev / (ev[0] + ev[1] + ev[2])
    out_v[...] = (wn[0] * emb_v[...] + wn[1] * s1_v[...]
                  + wn[2] * s2_v[...])


def _run_tc(emb_table, sinusoidal_1, sinusoidal_2, w3):
    return pl.pallas_call(
        _tc_body,
        grid=(_GT,),
        in_specs=[
            pl.BlockSpec(memory_space=pltpu.SMEM),
            pl.BlockSpec((_BT, _D), lambda i: (i, 0)),
            pl.BlockSpec((_BT, _D), lambda i: (i, 0)),
            pl.BlockSpec((_BT, _D), lambda i: (i, 0)),
        ],
        out_specs=pl.BlockSpec((_BT, _D), lambda i: (i, 0)),
        out_shape=jax.ShapeDtypeStruct((_SPLIT, _D), jnp.float32),
        compiler_params=pltpu.CompilerParams(
            dimension_semantics=("arbitrary",)),
    )(w3, emb_table, sinusoidal_1, sinusoidal_2)


@jax.jit
def _run(emb_table, sinusoidal_1, sinusoidal_2, w3, shift1):
    sc_out = _run_sc(emb_table, sinusoidal_1, sinusoidal_2, w3, shift1)
    tc_out = _run_tc(emb_table, sinusoidal_1, sinusoidal_2, w3)
    return lax.cond(
        shift1[0] == 0,
        lambda: lax.dynamic_update_slice(sc_out, tc_out, (0, 0)),
        lambda: sc_out,
    )


def kernel(emb_table, sinusoidal_1, sinusoidal_2, mixing_weights, seq_len):
    shift1 = jnp.reshape(jnp.asarray(seq_len, jnp.int32) - _N, (1,))
    return _run(emb_table, sinusoidal_1, sinusoidal_2,
                mixing_weights.astype(jnp.float32), shift1)


# speedup vs baseline: 1.1634x; 1.1221x over previous
import jax
import jax.numpy as jnp
from jax import lax
from jax.experimental import pallas as pl
from jax.experimental.pallas import tpu as pltpu
from jax.experimental.pallas import tpu_sc as plsc

_N = 8192
_D = 256
_SPLIT = 5120
_NC = 2
_NS = 16
_NW = _NC * _NS
_RW = (_N - _SPLIT) // _NW
_RW2 = _N // _NW
_B = 16
_RING = 6
_BT = 1280
_GT = _SPLIT // _BT


def _sc_body(emb_hbm, s1_hbm, s2_hbm, w_hbm, shift_hbm, out_hbm,
             buf_v, idxe_v, w_vt, sh_vt, isems, osems):
    wid = lax.axis_index("s") * _NC + lax.axis_index("c")
    base0 = _SPLIT + wid * _RW
    nb = _RW // _B
    assert nb <= _RING

    for b in range(nb):
        r0 = base0 + b * _B
        slot = b % _RING
        pltpu.async_copy(emb_hbm.at[pl.ds(r0, _B)],
                         buf_v.at[slot * 4 + 0], isems.at[slot])
        pltpu.async_copy(s1_hbm.at[pl.ds(r0, _B)],
                         buf_v.at[slot * 4 + 1], isems.at[slot])
        pltpu.async_copy(s2_hbm.at[pl.ds(r0, _B)],
                         buf_v.at[slot * 4 + 2], isems.at[slot])

    st1 = pltpu.async_copy(w_hbm, w_vt.at[pl.ds(0, 3)], osems.at[0])
    st2 = pltpu.async_copy(shift_hbm, sh_vt.at[pl.ds(0, 1)], osems.at[0])
    st1.wait()
    st2.wait()
    wload = w_vt[...]
    w0r = wload[0]
    w1r = wload[1]
    w2r = wload[2]
    shift = sh_vt[...][0]

    iot = lax.iota(jnp.int32, 16)
    wv = jnp.where(iot == 0, w0r,
                   jnp.where(iot == 1, w1r,
                             jnp.where(iot == 2, w2r, -1e30)))
    mx = jnp.maximum(jnp.maximum(w0r, w1r), w2r)
    ev = jnp.exp(wv - mx)
    ssum = ev[0] + ev[1] + ev[2]
    wn = ev / ssum
    w0 = wn[0]
    w1 = wn[1]
    w2 = wn[2]

    def idx_gather(r0, slot):
        for j in range(_B // 16):
            idxe_v[slot, pl.ds(j * 16, 16)] = jnp.clip(
                iot + (r0 + j * 16) + shift, 0, _N - 1)
        return pltpu.async_copy(emb_hbm.at[idxe_v.at[slot]],
                                buf_v.at[slot * 4 + 0], isems.at[slot])

    def wait_inputs(r0, slot):
        pltpu.make_async_copy(s1_hbm.at[pl.ds(r0, _B)],
                              buf_v.at[slot * 4 + 1], isems.at[slot]).wait()
        pltpu.make_async_copy(s1_hbm.at[pl.ds(r0, _B)],
                              buf_v.at[slot * 4 + 2], isems.at[slot]).wait()
        pltpu.make_async_copy(s1_hbm.at[pl.ds(r0, _B)],
                              buf_v.at[slot * 4 + 0], isems.at[slot]).wait()

    def compute(r0, slot):
        def comp(i, carry):
            for j in range(_D // 16):
                sl = pl.ds(j * 16, 16)
                buf_v[slot * 4 + 3, i, sl] = (
                    w0 * buf_v[slot * 4 + 0, i, sl]
                    + w1 * buf_v[slot * 4 + 1, i, sl]
                    + w2 * buf_v[slot * 4 + 2, i, sl])
            return carry

        lax.fori_loop(0, _B, comp, 0)
        return pltpu.async_copy(
            buf_v.at[slot * 4 + 3], out_hbm.at[pl.ds(r0, _B)],
            osems.at[slot])

    def main_blk(b, carry):
        slot = lax.rem(b, _RING)
        r0 = base0 + b * _B
        wait_inputs(r0, slot)
        compute(r0, slot)
        return carry

    lax.fori_loop(0, nb, main_blk, 0)

    def drain_wb(b, carry):
        slot = lax.rem(b, _RING)
        r0 = base0 + b * _B
        pltpu.make_async_copy(buf_v.at[slot * 4 + 3],
                              out_hbm.at[pl.ds(r0, _B)],
                              osems.at[slot]).wait()
        return carry

    lax.fori_loop(0, nb, drain_wb, 0)

    @pl.when(shift != 0)
    def _slow_full_coverage():
        base2 = wid * _RW2

        def blk(b, carry):
            r0 = base2 + b * _B
            pltpu.async_copy(s1_hbm.at[pl.ds(r0, _B)],
                             buf_v.at[1], isems.at[0])
            pltpu.async_copy(s2_hbm.at[pl.ds(r0, _B)],
                             buf_v.at[2], isems.at[0])
            idx_gather(r0, 0)
            wait_inputs(r0, 0)
            compute(r0, 0).wait()
            return carry

        lax.fori_loop(0, _RW2 // _B, blk, 0)


def _run_sc(emb_table, sinusoidal_1, sinusoidal_2, w3, shift1):
    f = pl.kernel(
        _sc_body,
        out_type=jax.ShapeDtypeStruct((_N, _D), jnp.float32),
        mesh=plsc.VectorSubcoreMesh(core_axis_name="c", subcore_axis_name="s"),
        scratch_types=[
            pltpu.VMEM((4 * _RING, _B, _D), jnp.float32),
            pltpu.VMEM((_RING, _B), jnp.int32),
            pltpu.VMEM((16,), jnp.float32),
            pltpu.VMEM((16,), jnp.int32),
            pltpu.SemaphoreType.DMA((_RING,)),
            pltpu.SemaphoreType.DMA((_RING,)),
        ],
    )
    return f(emb_table, sinusoidal_1, sinusoidal_2, w3, shift1)


def _tc_body(w_sm, emb_v, s1_v, s2_v, out_v):
    w0r = w_sm[0]
    w1r = w_sm[1]
    w2r = w_sm[2]
    iot = lax.iota(jnp.int32, 128)
    wv = jnp.where(iot == 0, w0r,
                   jnp.where(iot == 1, w1r,
                             jnp.where(iot == 2, w2r, -1e30)))
    mx = jnp.maximum(jnp.maximum(w0r, w1r), w2r)
    ev = jnp.exp(wv - mx)
    wn = ev / (ev[0] + ev[1] + ev[2])
    out_v[...] = (wn[0] * emb_v[...] + wn[1] * s1_v[...]
                  + wn[2] * s2_v[...])


def _run_tc(emb_table, sinusoidal_1, sinusoidal_2, w3):
    return pl.pallas_call(
        _tc_body,
        grid=(_GT,),
        in_specs=[
            pl.BlockSpec(memory_space=pltpu.SMEM),
            pl.BlockSpec((_BT, _D), lambda i: (i, 0)),
            pl.BlockSpec((_BT, _D), lambda i: (i, 0)),
            pl.BlockSpec((_BT, _D), lambda i: (i, 0)),
        ],
        out_specs=pl.BlockSpec((_BT, _D), lambda i: (i, 0)),
        out_shape=jax.ShapeDtypeStruct((_SPLIT, _D), jnp.float32),
        compiler_params=pltpu.CompilerParams(
            dimension_semantics=("arbitrary",)),
    )(w3, emb_table, sinusoidal_1, sinusoidal_2)


@jax.jit
def _run(emb_table, sinusoidal_1, sinusoidal_2, w3, shift1):
    sc_out = _run_sc(emb_table, sinusoidal_1, sinusoidal_2, w3, shift1)
    tc_out = _run_tc(emb_table, sinusoidal_1, sinusoidal_2, w3)
    return lax.cond(
        shift1[0] == 0,
        lambda: lax.dynamic_update_slice(sc_out, tc_out, (0, 0)),
        lambda: sc_out,
    )


def kernel(emb_table, sinusoidal_1, sinusoidal_2, mixing_weights, seq_len):
    shift1 = jnp.reshape(jnp.asarray(seq_len, jnp.int32) - _N, (1,))
    return _run(emb_table, sinusoidal_1, sinusoidal_2,
                mixing_weights.astype(jnp.float32), shift1)
